# Initial kernel scaffold; baseline (speedup 1.0000x reference)
import jax
import jax.numpy as jnp
from jax.experimental import pallas as pl


def kernel(vdata, edata, connectivity, cdata, metadata, params):
    raise NotImplementedError("write your pallas kernel here")


# speedup vs baseline: 3.8905x; 3.8905x over previous
import functools
import jax
import jax.numpy as jnp
from jax import lax
from jax.experimental import pallas as pl
from jax.experimental.pallas import tpu as pltpu
from jax.experimental.pallas import tpu_sc as plsc

N_NODES = 10000
N_EDGES = 160000
LAT = 128
HID = 16
CORE_STEPS = 3

BN = 1000
BE = 2000
NW = 32
EPW = N_EDGES // NW
NPT = N_NODES // 16
CH = 200
NCH = EPW // CH

_EPS = 1e-5


def _ln(h, g, bt):
    mu = jnp.mean(h, axis=-1, keepdims=True)
    var = jnp.mean((h - mu) * (h - mu), axis=-1, keepdims=True)
    return (h - mu) * lax.rsqrt(var + _EPS) * g + bt


def _dot(a, b):
    return jnp.dot(a, b, preferred_element_type=jnp.float32)


def _mlp_tail(h1pre, W2, b2, g, bt):
    h = jnp.maximum(h1pre, 0.0)
    h = jnp.maximum(_dot(h, W2) + b2, 0.0)
    return _ln(h, g, bt)


def _row_spec(b, d):
    return pl.BlockSpec((b, d), lambda i: (i, 0))


def _w_spec(shape):
    return pl.BlockSpec(shape, lambda i: tuple(0 for _ in shape))


def _enc_node_body(x, W1, b1, W2, b2, g, bt, Avs, Avd, Bv, Av0s, Av0d, Bv0,
                   ns_o, nd_o, pn_o, fs_o, fd_o, fn_o):
    h1 = _dot(x[...], W1[...]) + b1[...]
    v0 = _mlp_tail(h1, W2[...], b2[...], g[...], bt[...])
    fs = _dot(v0, Av0s[...])
    fd = _dot(v0, Av0d[...])
    fn = _dot(v0, Bv0[...])
    fs_o[...] = fs
    fd_o[...] = fd
    fn_o[...] = fn
    ns_o[...] = fs + _dot(v0, Avs[...])
    nd_o[...] = fd + _dot(v0, Avd[...])
    pn_o[...] = fn + _dot(v0, Bv[...])


def _enc_edge_body(x, W1, b1, W2, b2, g, bt, A0, A1, pe0_o, pec_o):
    h1 = _dot(x[...], W1[...]) + b1[...]
    e0 = _mlp_tail(h1, W2[...], b2[...], g[...], bt[...])
    pe0_o[...] = _dot(e0, A0[...])
    pec_o[...] = _dot(e0, A1[...])


def _edge_step_body(pe0, pec, gs, gd, b1, W2, b2, g, bt, Ae,
                    en_o, pec_o):
    h1 = pe0[...] + pec[...] + gs[...] + gd[...] + b1[...]
    en = _mlp_tail(h1, W2[...], b2[...], g[...], bt[...])
    en_o[...] = en
    pec_o[...] = _dot(en, Ae[...])


def _edge_last_body(pe0, pec, gs, gd, b1, W2, b2, g, bt,
                    dW1, db1, dW2, db2, dg, dbt, oW, ob,
                    en_o, eout_o):
    h1 = pe0[...] + pec[...] + gs[...] + gd[...] + b1[...]
    en = _mlp_tail(h1, W2[...], b2[...], g[...], bt[...])
    en_o[...] = en
    d1 = _dot(en, dW1[...]) + db1[...]
    dec = _mlp_tail(d1, dW2[...], db2[...], dg[...], dbt[...])
    eout_o[...] = _dot(dec, oW[...]) + ob[...]


def _node_step_body(pn, a0, a1, c0, c1, fs, fd, fn, b1, W2, b2, g, bt,
                    Bagg, Avs, Avd, Bv, ns_o, nd_o, pn_o):
    cm = jnp.maximum(c0[...] + c1[...], 1.0)[:, 0:1]
    agg = (a0[...] + a1[...]) / cm
    h1 = pn[...] + _dot(agg, Bagg[...]) + b1[...]
    vn = _mlp_tail(h1, W2[...], b2[...], g[...], bt[...])
    ns_o[...] = fs[...] + _dot(vn, Avs[...])
    nd_o[...] = fd[...] + _dot(vn, Avd[...])
    pn_o[...] = fn[...] + _dot(vn, Bv[...])


def _node_last_body(pn, a0, a1, c0, c1, b1, W2, b2, g, bt, Bagg,
                    dW1, db1, dW2, db2, dg, dbt, oW, ob, vout_o):
    cm = jnp.maximum(c0[...] + c1[...], 1.0)[:, 0:1]
    agg = (a0[...] + a1[...]) / cm
    h1 = pn[...] + _dot(agg, Bagg[...]) + b1[...]
    vn = _mlp_tail(h1, W2[...], b2[...], g[...], bt[...])
    d1 = _dot(vn, dW1[...]) + db1[...]
    dec = _mlp_tail(d1, dW2[...], db2[...], dg[...], dbt[...])
    vout_o[...] = _dot(dec, oW[...]) + ob[...]


def _tc_call(body, n_rows, blk, in_specs, out_shapes, out_specs, interpret=False):
    return pl.pallas_call(
        body,
        grid=(n_rows // blk,),
        in_specs=in_specs,
        out_specs=out_specs,
        out_shape=out_shapes,
        interpret=interpret,
    )


@functools.cache
def _sc_kernels():
    mesh = plsc.VectorSubcoreMesh(core_axis_name="c", subcore_axis_name="s")
    cp = pltpu.CompilerParams(use_tc_tiling_on_sc=False)

    @functools.partial(
        pl.kernel,
        out_type=[jax.ShapeDtypeStruct((N_EDGES, HID), jnp.float32),
                  jax.ShapeDtypeStruct((N_EDGES, HID), jnp.float32)],
        mesh=mesh,
        compiler_params=cp,
        scratch_types=[pltpu.VMEM((EPW,), jnp.int32),
                       pltpu.VMEM((EPW, HID), jnp.float32),
                       pltpu.SemaphoreType.DMA],
    )
    def sc_gather(ns_h, nd_h, src_h, dst_h, gs_h, gd_h, idx_v, rows_v, sem):
        wid = lax.axis_index("s") * 2 + lax.axis_index("c")
        base = wid * EPW
        sl = pl.ds(base, EPW)
        pltpu.sync_copy(src_h.at[sl], idx_v)
        pltpu.async_copy(ns_h.at[idx_v], rows_v, sem).wait()
        pltpu.sync_copy(rows_v, gs_h.at[sl])
        pltpu.sync_copy(dst_h.at[sl], idx_v)
        pltpu.async_copy(nd_h.at[idx_v], rows_v, sem).wait()
        pltpu.sync_copy(rows_v, gd_h.at[sl])

    @functools.partial(
        pl.kernel,
        out_type=[jax.ShapeDtypeStruct((2, N_NODES, HID), jnp.float32)],
        mesh=mesh,
        compiler_params=cp,
        scratch_types=[pltpu.VMEM((EPW,), jnp.int32),
                       pltpu.VMEM((EPW, HID), jnp.float32),
                       pltpu.VMEM_SHARED((N_NODES, HID), jnp.float32)],
    )
    def sc_count(pa_h, dst_h, zeros_h, out_h, idx_v, pa_v, acc):
        sid = lax.axis_index("s")
        cid = lax.axis_index("c")
        base = (sid * 2 + cid) * EPW
        nsl = pl.ds(sid * NPT, NPT)
        pltpu.sync_copy(zeros_h.at[nsl], acc.at[nsl])
        plsc.subcore_barrier()
        pltpu.sync_copy(dst_h.at[pl.ds(base, EPW)], idx_v)
        pltpu.sync_copy(pa_h.at[pl.ds(base, EPW)], pa_v)
        pltpu.sync_copy(pa_v, acc.at[idx_v], add=True)
        plsc.subcore_barrier()
        pltpu.sync_copy(acc.at[nsl], out_h.at[cid, nsl])

    @functools.partial(
        pl.kernel,
        out_type=[jax.ShapeDtypeStruct((2, N_NODES, LAT), jnp.float32)],
        mesh=mesh,
        compiler_params=cp,
        scratch_types=[pltpu.VMEM((NCH, CH), jnp.int32),
                       pltpu.VMEM((CH, LAT), jnp.float32),
                       pltpu.VMEM_SHARED((N_NODES, LAT), jnp.float32)],
    )
    def sc_scatter(pa_h, dst_h, zeros_h, out_h, idx2, pa_v, acc):
        sid = lax.axis_index("s")
        cid = lax.axis_index("c")
        base = (sid * 2 + cid) * EPW
        nsl = pl.ds(sid * NPT, NPT)
        pltpu.sync_copy(zeros_h.at[nsl], acc.at[nsl])
        plsc.subcore_barrier()
        for j in range(NCH):
            pltpu.sync_copy(dst_h.at[pl.ds(base + j * CH, CH)], idx2.at[j])
            pltpu.sync_copy(pa_h.at[pl.ds(base + j * CH, CH)], pa_v)
            pltpu.sync_copy(pa_v, acc.at[idx2.at[j]], add=True)
        plsc.subcore_barrier()
        pltpu.sync_copy(acc.at[nsl], out_h.at[cid, nsl])

    return sc_gather, sc_count, sc_scatter


def _forward_impl(vdata, edata, connectivity, params, interpret=False):
    f32 = jnp.float32
    src = connectivity[0]
    dst = connectivity[1]

    pce, pcn = params["core_edge"], params["core_node"]
    W1ce, W1cn = pce["W1"], pcn["W1"]
    A_e0, A_e = W1ce[0:LAT], W1ce[LAT:2 * LAT]
    A_v0s, A_vs = W1ce[2 * LAT:3 * LAT], W1ce[3 * LAT:4 * LAT]
    A_v0d, A_vd = W1ce[4 * LAT:5 * LAT], W1ce[5 * LAT:6 * LAT]
    B_v0, B_v, B_agg = W1cn[0:LAT], W1cn[LAT:2 * LAT], W1cn[2 * LAT:3 * LAT]

    def r2(x):
        return x.reshape(1, -1)

    en_, ee_ = params["enc_node"], params["enc_edge"]
    dn_, de_ = params["dec_node"], params["dec_edge"]

    w16 = _w_spec((LAT, HID))
    w1h = _w_spec((1, HID))
    w1l = _w_spec((1, LAT))

    ns, nd, pn, f_s, f_d, f_n = _tc_call(
        _enc_node_body, N_NODES, BN,
        in_specs=[_row_spec(BN, LAT), _w_spec((LAT, HID)), w1h,
                  _w_spec((HID, LAT)), w1l, w1l, w1l,
                  w16, w16, w16, w16, w16, w16],
        out_shapes=[jax.ShapeDtypeStruct((N_NODES, HID), f32)] * 6,
        out_specs=[_row_spec(BN, HID)] * 6,
        interpret=interpret,
    )(vdata, en_["W1"], r2(en_["b1"]), en_["W2"], r2(en_["b2"]),
      r2(en_["g"]), r2(en_["bt"]),
      A_vs, A_vd, B_v, A_v0s, A_v0d, B_v0)

    pe0, pec = _tc_call(
        _enc_edge_body, N_EDGES, BE,
        in_specs=[_row_spec(BE, HID), _w_spec((HID, HID)), w1h,
                  _w_spec((HID, LAT)), w1l, w1l, w1l, w16, w16],
        out_shapes=[jax.ShapeDtypeStruct((N_EDGES, HID), f32)] * 2,
        out_specs=[_row_spec(BE, HID)] * 2,
        interpret=interpret,
    )(edata, ee_["W1"], r2(ee_["b1"]), ee_["W2"], r2(ee_["b2"]),
      r2(ee_["g"]), r2(ee_["bt"]), A_e0, A_e)

    zeros_nh = jnp.zeros((N_NODES, HID), f32)
    zeros_nl = jnp.zeros((N_NODES, LAT), f32)
    ones_e = jnp.ones((N_EDGES, HID), f32)

    if interpret:
        def do_gather(ns_, nd_):
            return jnp.take(ns_, src, axis=0), jnp.take(nd_, dst, axis=0)

        def _seg(x):
            s = jax.ops.segment_sum(x, dst, num_segments=N_NODES)
            return jnp.stack([s, jnp.zeros_like(s)])

        do_count = _seg
        do_scatter = _seg
    else:
        sc_gather, sc_count, sc_scatter = _sc_kernels()

        def _unwrap(out):
            if isinstance(out, (list, tuple)):
                out = out[0]
            return out

        def do_gather(ns_, nd_):
            return sc_gather(ns_, nd_, src, dst)

        def do_count(x):
            return _unwrap(sc_count(x, dst, zeros_nh))

        def do_scatter(x):
            return _unwrap(sc_scatter(x, dst, zeros_nl))

    cntp = do_count(ones_e)

    for t in range(CORE_STEPS):
        gs, gd = do_gather(ns, nd)
        last = t == CORE_STEPS - 1
        if not last:
            en, pec = _tc_call(
                _edge_step_body, N_EDGES, BE,
                in_specs=[_row_spec(BE, HID)] * 4 + [w1h, _w_spec((HID, LAT)),
                          w1l, w1l, w1l, w16],
                out_shapes=[jax.ShapeDtypeStruct((N_EDGES, LAT), f32),
                            jax.ShapeDtypeStruct((N_EDGES, HID), f32)],
                out_specs=[_row_spec(BE, LAT), _row_spec(BE, HID)],
                interpret=interpret,
            )(pe0, pec, gs, gd, r2(pce["b1"]), pce["W2"], r2(pce["b2"]),
              r2(pce["g"]), r2(pce["bt"]), A_e)
        else:
            en, e_out = _tc_call(
                _edge_last_body, N_EDGES, BE,
                in_specs=[_row_spec(BE, HID)] * 4 + [w1h, _w_spec((HID, LAT)),
                          w1l, w1l, w1l,
                          _w_spec((LAT, HID)), w1h, _w_spec((HID, LAT)),
                          w1l, w1l, w1l, _w_spec((LAT, LAT)), w1l],
                out_shapes=[jax.ShapeDtypeStruct((N_EDGES, LAT), f32)] * 2,
                out_specs=[_row_spec(BE, LAT)] * 2,
                interpret=interpret,
            )(pe0, pec, gs, gd, r2(pce["b1"]), pce["W2"], r2(pce["b2"]),
              r2(pce["g"]), r2(pce["bt"]),
              de_["W1"], r2(de_["b1"]), de_["W2"], r2(de_["b2"]),
              r2(de_["g"]), r2(de_["bt"]),
              params["dec_edge_out_W"], r2(params["dec_edge_out_b"]))

        aggp = do_scatter(en)

        if not last:
            ns, nd, pn = _tc_call(
                _node_step_body, N_NODES, BN,
                in_specs=[_row_spec(BN, HID), _row_spec(BN, LAT),
                          _row_spec(BN, LAT), _row_spec(BN, HID),
                          _row_spec(BN, HID), _row_spec(BN, HID),
                          _row_spec(BN, HID), _row_spec(BN, HID),
                          w1h, _w_spec((HID, LAT)), w1l, w1l, w1l,
                          w16, w16, w16, w16],
                out_shapes=[jax.ShapeDtypeStruct((N_NODES, HID), f32)] * 3,
                out_specs=[_row_spec(BN, HID)] * 3,
                interpret=interpret,
            )(pn, aggp[0], aggp[1], cntp[0], cntp[1], f_s, f_d, f_n,
              r2(pcn["b1"]), pcn["W2"], r2(pcn["b2"]), r2(pcn["g"]),
              r2(pcn["bt"]), B_agg, A_vs, A_vd, B_v)
        else:
            v_out = _tc_call(
                _node_last_body, N_NODES, BN,
                in_specs=[_row_spec(BN, HID), _row_spec(BN, LAT),
                          _row_spec(BN, LAT), _row_spec(BN, HID),
                          _row_spec(BN, HID),
                          w1h, _w_spec((HID, LAT)), w1l, w1l, w1l, w16,
                          _w_spec((LAT, HID)), w1h, _w_spec((HID, LAT)),
                          w1l, w1l, w1l, _w_spec((LAT, LAT)), w1l],
                out_shapes=jax.ShapeDtypeStruct((N_NODES, LAT), f32),
                out_specs=_row_spec(BN, LAT),
                interpret=interpret,
            )(pn, aggp[0], aggp[1], cntp[0], cntp[1],
              r2(pcn["b1"]), pcn["W2"], r2(pcn["b2"]), r2(pcn["g"]),
              r2(pcn["bt"]), B_agg,
              dn_["W1"], r2(dn_["b1"]), dn_["W2"], r2(dn_["b2"]),
              r2(dn_["g"]), r2(dn_["bt"]),
              params["dec_node_out_W"], r2(params["dec_node_out_b"]))

    return (v_out, e_out)


def kernel(vdata, edata, connectivity, cdata, metadata, params):
    return _forward_impl(vdata, edata, connectivity, params)
